# baseline (speedup 1.0000x reference)
import jax
import jax.numpy as jnp
from jax.experimental import pallas as pl
from jax.experimental.pallas import tpu as pltpu

_TN = 512


def _moe_kernel(x_ref, wr_ref, br_ref, w2_ref, be_ref, out_ref, a_ref):
    xv = x_ref[...]
    D = xv.shape[1]

    logits = jnp.dot(xv, wr_ref[...], preferred_element_type=jnp.float32)
    logits = logits + br_ref[...]
    m = jnp.max(logits, axis=1, keepdims=True)
    ex = jnp.exp(logits - m)
    g = ex / jnp.sum(ex, axis=1, keepdims=True)
    E = g.shape[1]

    for e in range(E):
        a_ref[:, e * D:(e + 1) * D] = (xv * g[:, e:e + 1]).astype(jnp.bfloat16)

    y = jnp.dot(a_ref[...], w2_ref[...], preferred_element_type=jnp.float32)
    out_ref[...] = y + jnp.dot(g, be_ref[...],
                               preferred_element_type=jnp.float32)


def kernel(x, Wr, br, We, be):
    N, D = x.shape
    E = We.shape[0]
    br2 = br.reshape(1, E)
    W2 = We.reshape(E * D, D).astype(jnp.bfloat16)
    return pl.pallas_call(
        _moe_kernel,
        grid=(N // _TN,),
        in_specs=[
            pl.BlockSpec((_TN, D), lambda i: (i, 0)),
            pl.BlockSpec((D, E), lambda i: (0, 0)),
            pl.BlockSpec((1, E), lambda i: (0, 0)),
            pl.BlockSpec((E * D, D), lambda i: (0, 0)),
            pl.BlockSpec((E, D), lambda i: (0, 0)),
        ],
        out_specs=pl.BlockSpec((_TN, D), lambda i: (i, 0)),
        out_shape=jax.ShapeDtypeStruct((N, D), jnp.float32),
        scratch_shapes=[pltpu.VMEM((_TN, E * D), jnp.bfloat16)],
        compiler_params=pltpu.CompilerParams(
            dimension_semantics=("arbitrary",)),
    )(x, Wr, br2, W2, be)


# speedup vs baseline: 1.1233x; 1.1233x over previous
import jax
import jax.numpy as jnp
from jax.experimental import pallas as pl
from jax.experimental.pallas import tpu as pltpu

_TN = 512


def _moe_kernel(x_ref, wr_ref, br_ref, w2_ref, be_ref, out_ref,
                a_ref, w2b_ref):
    s = pl.program_id(0)
    E = wr_ref.shape[1]
    D = x_ref.shape[1]

    @pl.when(s < E)
    def _warm():
        w2b_ref[pl.ds(s * D, D), :] = w2_ref[...].astype(jnp.bfloat16)

    @pl.when(s >= E)
    def _compute():
        xv = x_ref[...]
        logits = jnp.dot(xv, wr_ref[...], preferred_element_type=jnp.float32)
        logits = logits + br_ref[...]
        m = jnp.max(logits, axis=1, keepdims=True)
        ex = jnp.exp(logits - m)
        g = ex / jnp.sum(ex, axis=1, keepdims=True)

        for e in range(E):
            a_ref[:, e * D:(e + 1) * D] = (
                xv * g[:, e:e + 1]).astype(jnp.bfloat16)

        y = jnp.dot(a_ref[...], w2b_ref[...],
                    preferred_element_type=jnp.float32)
        out_ref[...] = y + jnp.dot(g, be_ref[...],
                                   preferred_element_type=jnp.float32)


def kernel(x, Wr, br, We, be):
    N, D = x.shape
    E = We.shape[0]
    br2 = br.reshape(1, E)
    W2 = We.reshape(E * D, D)
    nt = N // _TN
    return pl.pallas_call(
        _moe_kernel,
        grid=(E + nt,),
        in_specs=[
            pl.BlockSpec((_TN, D), lambda s: (jnp.maximum(s - E, 0), 0)),
            pl.BlockSpec((D, E), lambda s: (0, 0)),
            pl.BlockSpec((1, E), lambda s: (0, 0)),
            pl.BlockSpec((D, D), lambda s: (jnp.minimum(s, E - 1), 0)),
            pl.BlockSpec((E, D), lambda s: (0, 0)),
        ],
        out_specs=pl.BlockSpec((_TN, D), lambda s: (jnp.maximum(s - E, 0), 0)),
        out_shape=jax.ShapeDtypeStruct((N, D), jnp.float32),
        scratch_shapes=[
            pltpu.VMEM((_TN, E * D), jnp.bfloat16),
            pltpu.VMEM((E * D, D), jnp.bfloat16),
        ],
        compiler_params=pltpu.CompilerParams(
            dimension_semantics=("arbitrary",)),
    )(x, Wr, br2, W2, be)
